# baseline (speedup 1.0000x reference)
import functools

import jax
import jax.numpy as jnp
from jax import lax
from jax.experimental import pallas as pl
from jax.experimental.pallas import tpu as pltpu
from jax.experimental.pallas import tpu_sc as plsc

_D = 64
_K = 1024
_N = 16 * 1024
_BLK = 512
_COMMIT = 0.25


def _vq_block(z_ref, e_ref, idx_ref, loss_ref):
    i = pl.program_id(0)
    zb = z_ref[...]
    emb = e_ref[...]
    dots = jnp.dot(zb, emb, preferred_element_type=jnp.float32)
    rown = jnp.sum(zb * zb, axis=1, keepdims=True)
    enorm = jnp.sum(emb * emb, axis=0, keepdims=True)
    dist = (rown - 2.0 * dots) + enorm
    minv = jnp.min(dist, axis=1, keepdims=True)
    iota = lax.broadcasted_iota(jnp.int32, (_BLK, _K), 1)
    idx = jnp.min(jnp.where(dist == minv, iota, _K), axis=1)
    idx_ref[0, 0, :] = idx

    @pl.when(i == 0)
    def _init():
        loss_ref[0, 0] = 0.0

    loss_ref[0, 0] += jnp.sum(minv)

    @pl.when(i == pl.num_programs(0) - 1)
    def _fin():
        loss_ref[0, 0] = loss_ref[0, 0] * (_COMMIT / (_N * _D))


_SC_INFO = plsc.get_sparse_core_info()
_NC = _SC_INFO.num_cores
_NS = _SC_INFO.num_subcores
_NW = _NC * _NS
_ROWS_PER_W = _N // _NW


@functools.partial(
    pl.kernel,
    mesh=plsc.VectorSubcoreMesh(core_axis_name="c", subcore_axis_name="s"),
    compiler_params=pltpu.CompilerParams(use_tc_tiling_on_sc=False),
    out_type=jax.ShapeDtypeStruct((_N, _D), jnp.float32),
    scratch_types=[
        pltpu.VMEM((_ROWS_PER_W,), jnp.int32),
        pltpu.VMEM((_ROWS_PER_W, _D), jnp.float32),
        pltpu.SemaphoreType.DMA,
    ],
)
def _sc_gather(table_hbm, idx_hbm, out_hbm, idx_v, rows_v, sem):
    wid = lax.axis_index("s") * _NC + lax.axis_index("c")
    base = wid * _ROWS_PER_W
    pltpu.sync_copy(idx_hbm.at[pl.ds(base, _ROWS_PER_W)], idx_v)
    pltpu.async_copy(table_hbm.at[idx_v], rows_v, sem).wait()
    pltpu.sync_copy(rows_v, out_hbm.at[pl.ds(base, _ROWS_PER_W)])


def kernel(z, embeddings, is_training):
    zf = z.reshape(-1, _D)
    nblk = _N // _BLK
    idx3, loss = pl.pallas_call(
        _vq_block,
        grid=(nblk,),
        in_specs=[
            pl.BlockSpec((_BLK, _D), lambda i: (i, 0)),
            pl.BlockSpec((_D, _K), lambda i: (0, 0)),
        ],
        out_specs=[
            pl.BlockSpec((1, 1, _BLK), lambda i: (i, 0, 0)),
            pl.BlockSpec(block_shape=(1, 1), index_map=lambda i: (0, 0),
                         memory_space=pltpu.SMEM),
        ],
        out_shape=[
            jax.ShapeDtypeStruct((nblk, 1, _BLK), jnp.int32),
            jax.ShapeDtypeStruct((1, 1), jnp.float32),
        ],
    )(zf, embeddings)
    idx = idx3.reshape(-1)
    q = _sc_gather(embeddings.T, idx)
    return q.reshape(z.shape), loss[0, 0], idx


# speedup vs baseline: 1.4196x; 1.4196x over previous
import jax
import jax.numpy as jnp
from jax import lax
from jax.experimental import pallas as pl
from jax.experimental.pallas import tpu as pltpu

_D = 64
_K = 1024
_BLK = 512
_COMMIT = 0.25


def _vq_block(z_ref, e_ref, q_ref, idx_ref, loss_ref):
    i = pl.program_id(0)
    zb = z_ref[...]
    emb = e_ref[...]
    dots2 = jnp.dot(zb, emb + emb, preferred_element_type=jnp.float32)
    rown = jnp.sum(zb * zb, axis=1, keepdims=True)
    enorm = jnp.sum(emb * emb, axis=0, keepdims=True)
    dist = (rown - dots2) + enorm
    minv = jnp.min(dist, axis=1, keepdims=True)
    iota = lax.broadcasted_iota(jnp.int32, (_BLK, _K), 1)
    idx = jnp.min(jnp.where(dist == minv, iota, _K), axis=1)
    idx_ref[0, 0, :] = idx
    onehot = (iota == idx[:, None]).astype(jnp.float32)
    q_ref[...] = lax.dot_general(
        onehot, emb, (((1,), (1,)), ((), ())),
        preferred_element_type=jnp.float32)

    @pl.when(i == 0)
    def _init():
        loss_ref[0, 0] = 0.0

    loss_ref[0, 0] += jnp.sum(minv)

    @pl.when(i == pl.num_programs(0) - 1)
    def _fin():
        loss_ref[0, 0] = loss_ref[0, 0] * (_COMMIT / (16 * 1024 * _D))


def kernel(z, embeddings, is_training):
    zf = z.reshape(-1, _D)
    n = zf.shape[0]
    nblk = n // _BLK
    q, idx3, loss = pl.pallas_call(
        _vq_block,
        grid=(nblk,),
        in_specs=[
            pl.BlockSpec((_BLK, _D), lambda i: (i, 0)),
            pl.BlockSpec((_D, _K), lambda i: (0, 0)),
        ],
        out_specs=[
            pl.BlockSpec((_BLK, _D), lambda i: (i, 0)),
            pl.BlockSpec((1, 1, _BLK), lambda i: (i, 0, 0)),
            pl.BlockSpec(block_shape=(1, 1), index_map=lambda i: (0, 0),
                         memory_space=pltpu.SMEM),
        ],
        out_shape=[
            jax.ShapeDtypeStruct((n, _D), jnp.float32),
            jax.ShapeDtypeStruct((nblk, 1, _BLK), jnp.int32),
            jax.ShapeDtypeStruct((1, 1), jnp.float32),
        ],
    )(zf, embeddings)
    return q.reshape(z.shape), loss[0, 0], idx3.reshape(-1)
